# Initial kernel scaffold; baseline (speedup 1.0000x reference)
import jax
import jax.numpy as jnp
from jax.experimental import pallas as pl


def kernel(x, arm):
    raise NotImplementedError("write your pallas kernel here")


# speedup vs baseline: 2.2780x; 2.2780x over previous
import functools

import jax
import jax.numpy as jnp
from jax import lax
from jax.experimental import pallas as pl
from jax.experimental.pallas import tpu as pltpu
from jax.experimental.pallas import tpu_sc as plsc

B = 8
R = 1000
A = 4
T = 20
F = 32
D = T * F

NC = 2
NS = 16
NW = NC * NS

ROWS_PER_W = (B * R * A) // NW
CHUNK = 80
TAIL = 48
_CHUNKS = [(i * CHUNK, CHUNK) for i in range(ROWS_PER_W // CHUNK)]
_CHUNKS.append((ROWS_PER_W - TAIL, TAIL))


def _make_gather():
    mesh = plsc.VectorSubcoreMesh(core_axis_name="c", subcore_axis_name="s")

    @functools.partial(
        pl.kernel,
        mesh=mesh,
        out_type=jax.ShapeDtypeStruct((NW * ROWS_PER_W, D), jnp.float32),
        scratch_types=[
            pltpu.VMEM((CHUNK,), jnp.int32),
            pltpu.VMEM((CHUNK,), jnp.int32),
            pltpu.VMEM((TAIL,), jnp.int32),
            pltpu.VMEM((CHUNK, D), jnp.float32),
            pltpu.VMEM((CHUNK, D), jnp.float32),
            pltpu.SemaphoreType.DMA,
            pltpu.SemaphoreType.DMA,
            pltpu.SemaphoreType.DMA,
            pltpu.SemaphoreType.DMA,
        ],
    )
    def gather_kernel(x_hbm, arm_hbm, out_hbm,
                      idx_a, idx_b, idx_t, rows_a, rows_b,
                      gs_a, gs_b, ss_a, ss_b):
        wid = lax.axis_index("s") * NC + lax.axis_index("c")
        b = wid // (R * A // ROWS_PER_W)
        base = wid * ROWS_PER_W
        idx = (idx_a, idx_b)
        rows = (rows_a, rows_b)
        gs = (gs_a, gs_b)
        ss = (ss_a, ss_b)

        def load_and_gather(j):
            off, sz = _CHUNKS[j]
            k = j % 2
            i_ref = idx[k] if sz == CHUNK else idx_t
            r_ref = rows[k] if sz == CHUNK else rows[k].at[pl.ds(0, sz)]
            pltpu.sync_copy(arm_hbm.at[pl.ds(base + off, sz)], i_ref)
            return pltpu.async_copy(x_hbm.at[b].at[i_ref], r_ref, gs[k])

        def store(j):
            off, sz = _CHUNKS[j]
            k = j % 2
            r_ref = rows[k] if sz == CHUNK else rows[k].at[pl.ds(0, sz)]
            return pltpu.async_copy(
                r_ref, out_hbm.at[pl.ds(base + off, sz)], ss[k])

        gat = [None, None]
        scat = [None, None]
        gat[0] = load_and_gather(0)
        for j in range(len(_CHUNKS)):
            cur, nxt = j % 2, (j + 1) % 2
            if j + 1 < len(_CHUNKS):
                if scat[nxt] is not None:
                    scat[nxt].wait()
                gat[nxt] = load_and_gather(j + 1)
            gat[cur].wait()
            scat[cur] = store(j)
        scat[0].wait()
        scat[1].wait()

    return gather_kernel


def kernel(x, arm):
    xf = x.reshape(B, R, D)
    af = arm.reshape(NW * ROWS_PER_W)
    out = _make_gather()(xf, af)
    return out.reshape(B, R, A, T, F)


# speedup vs baseline: 3.0752x; 1.3499x over previous
import functools

import jax
import jax.numpy as jnp
from jax import lax
from jax.experimental import pallas as pl
from jax.experimental.pallas import tpu as pltpu
from jax.experimental.pallas import tpu_sc as plsc

B = 8
R = 1000
A = 4
T = 20
F = 32

NC = 2
NS = 16
NW = NC * NS

PAIRS = B * T
PPW = PAIRS // NW
NRCH = R // 16 + 1
L = 16


def _make_gather():
    mesh = plsc.VectorSubcoreMesh(core_axis_name="c", subcore_axis_name="s")

    @functools.partial(
        pl.kernel,
        mesh=mesh,
        compiler_params=pltpu.CompilerParams(
            use_tc_tiling_on_sc=False, needs_layout_passes=False),
        out_type=jax.ShapeDtypeStruct((B, A, T, F, R), jnp.float32),
        scratch_types=[
            pltpu.VMEM((A, R), jnp.int32),
            pltpu.VMEM((F, R), jnp.float32),
            pltpu.VMEM((F, R), jnp.float32),
            pltpu.SemaphoreType.DMA,
        ],
    )
    def gather_kernel(xt_hbm, armt_hbm, out_hbm, perms, in_slab, out_slab,
                      sem):
        wid = lax.axis_index("s") * NC + lax.axis_index("c")
        pair0 = wid * PPW
        b = pair0 // T
        pltpu.sync_copy(armt_hbm.at[pl.ds(b * A, A)], perms)
        for p in range(PPW):
            t = pair0 % T + p
            pltpu.sync_copy(xt_hbm.at[b, t], in_slab)
            for a in range(A):
                def rchunk(i, carry, a=a):
                    rb = pl.multiple_of(i * L, L)
                    pv = perms[a, pl.ds(rb, L)]
                    for f in range(F):
                        row = jnp.full((L,), f, jnp.int32)
                        vals = plsc.load_gather(in_slab, [row, pv])
                        out_slab[f, pl.ds(rb, L)] = vals
                    return carry
                lax.fori_loop(0, R // L, rchunk, 0)
                pv = perms[a, pl.ds(R - L, L)]
                for f in range(F):
                    row = jnp.full((L,), f, jnp.int32)
                    vals = plsc.load_gather(in_slab, [row, pv])
                    out_slab[f, pl.ds(R - L, L)] = vals
                pltpu.async_copy(out_slab, out_hbm.at[b, a, t], sem).wait()

    return gather_kernel


def kernel(x, arm):
    xt = jnp.transpose(x, (0, 2, 3, 1))
    armt = jnp.transpose(arm, (0, 2, 1)).reshape(B * A, R)
    outp = _make_gather()(xt, armt)
    return jnp.transpose(outp, (0, 4, 1, 2, 3))


# speedup vs baseline: 5.1423x; 1.6722x over previous
import functools

import jax
import jax.numpy as jnp
from jax import lax
from jax.experimental import pallas as pl
from jax.experimental.pallas import tpu as pltpu
from jax.experimental.pallas import tpu_sc as plsc

B = 8
R = 1000
A = 4
T = 20
F = 32

NC = 2
NS = 16
NW = NC * NS

PAIRS = B * T
PPW = PAIRS // NW
NRCH = R // 16 + 1
L = 16


def _make_gather():
    mesh = plsc.VectorSubcoreMesh(core_axis_name="c", subcore_axis_name="s")

    @functools.partial(
        pl.kernel,
        mesh=mesh,
        compiler_params=pltpu.CompilerParams(
            use_tc_tiling_on_sc=False, needs_layout_passes=False),
        out_type=jax.ShapeDtypeStruct((B, A, T, F, R), jnp.float32),
        scratch_types=[
            pltpu.VMEM((A, R), jnp.int32),
            pltpu.VMEM((F, R), jnp.float32),
            pltpu.VMEM((F, R), jnp.float32),
            pltpu.SemaphoreType.DMA,
        ],
    )
    def gather_kernel(xt_hbm, armt_hbm, out_hbm, perms, in_slab, out_slab,
                      sem):
        wid = lax.axis_index("s") * NC + lax.axis_index("c")
        pair0 = wid * PPW
        b = pair0 // T
        pltpu.sync_copy(armt_hbm.at[pl.ds(b * A, A)], perms)

        def slab_body(p, carry):
            t = pair0 % T + p
            pltpu.sync_copy(xt_hbm.at[b, t], in_slab)
            for a in range(A):
                @plsc.parallel_loop(0, (R // L) * L, step=L)
                def rchunk(rb, a=a):
                    rb = pl.multiple_of(rb, L)
                    pv = perms[a, pl.ds(rb, L)]
                    for f in range(F):
                        row = jnp.full((L,), f, jnp.int32)
                        vals = plsc.load_gather(in_slab, [row, pv])
                        out_slab[f, pl.ds(rb, L)] = vals
                pv = perms[a, pl.ds(R - L, L)]
                for f in range(F):
                    row = jnp.full((L,), f, jnp.int32)
                    vals = plsc.load_gather(in_slab, [row, pv])
                    out_slab[f, pl.ds(R - L, L)] = vals
                pltpu.async_copy(out_slab, out_hbm.at[b, a, t], sem).wait()
            return carry

        lax.fori_loop(0, PPW, slab_body, 0)

    return gather_kernel


def kernel(x, arm):
    xt = jnp.transpose(x, (0, 2, 3, 1))
    armt = jnp.transpose(arm, (0, 2, 1)).reshape(B * A, R)
    outp = _make_gather()(xt, armt)
    return jnp.transpose(outp, (0, 4, 1, 2, 3))
